# baseline (device time: 14725 ns/iter reference)
import jax
import jax.numpy as jnp
from jax import lax
from jax.experimental import pallas as pl
from jax.experimental.pallas import tpu as pltpu

N_CHUNK = 8


def kernel(x):
    m, n = x.shape
    half = m // 2
    rows_c = half // N_CHUNK

    def body(x_ref, out_ref, rx_ref, ry_ref,
             sx_send, sx_recv, sy_send, sy_recv):
        mx = lax.axis_index("x")
        my = lax.axis_index("y")
        mz = lax.axis_index("z")
        h = my % 2
        xp = (1 - mx, my, mz)
        yp = (mx, my + 1 - 2 * h, mz)

        barrier_sem = pltpu.get_barrier_semaphore()
        for nbr in (xp, yp):
            pl.semaphore_signal(
                barrier_sem, inc=1, device_id=nbr,
                device_id_type=pl.DeviceIdType.MESH,
            )
        pl.semaphore_wait(barrier_sem, 2)

        def my_rows(c):
            return pl.ds(h * half + c * rows_c, rows_c)

        def other_rows(c):
            return pl.ds((1 - h) * half + c * rows_c, rows_c)

        xr = []
        for c in range(N_CHUNK):
            r = pltpu.make_async_remote_copy(
                src_ref=x_ref.at[my_rows(c), :],
                dst_ref=rx_ref.at[c],
                send_sem=sx_send.at[c],
                recv_sem=sx_recv.at[c],
                device_id=xp,
                device_id_type=pl.DeviceIdType.MESH,
            )
            r.start()
            xr.append(r)

        yr = []
        for c in range(N_CHUNK):
            xr[c].wait_recv()
            out_ref[my_rows(c), :] = x_ref[my_rows(c), :] + rx_ref[c]
            r = pltpu.make_async_remote_copy(
                src_ref=out_ref.at[my_rows(c), :],
                dst_ref=ry_ref.at[c],
                send_sem=sy_send.at[c],
                recv_sem=sy_recv.at[c],
                device_id=yp,
                device_id_type=pl.DeviceIdType.MESH,
            )
            r.start()
            yr.append(r)

        for c in range(N_CHUNK):
            yr[c].wait_recv()
            out_ref[other_rows(c), :] = ry_ref[c]

        for c in range(N_CHUNK):
            xr[c].wait_send()
            yr[c].wait_send()

    return pl.pallas_call(
        body,
        out_shape=jax.ShapeDtypeStruct((m, n), x.dtype),
        in_specs=[pl.BlockSpec(memory_space=pltpu.VMEM)],
        out_specs=pl.BlockSpec(memory_space=pltpu.VMEM),
        scratch_shapes=[
            pltpu.VMEM((N_CHUNK, rows_c, n), x.dtype),
            pltpu.VMEM((N_CHUNK, rows_c, n), x.dtype),
            pltpu.SemaphoreType.DMA((N_CHUNK,)),
            pltpu.SemaphoreType.DMA((N_CHUNK,)),
            pltpu.SemaphoreType.DMA((N_CHUNK,)),
            pltpu.SemaphoreType.DMA((N_CHUNK,)),
        ],
        compiler_params=pltpu.CompilerParams(collective_id=0),
    )(x)
